# baseline (speedup 1.0000x reference)
import functools

import jax
import jax.numpy as jnp
from jax import lax
from jax.experimental import pallas as pl
from jax.experimental.pallas import tpu as pltpu
from jax.experimental.pallas import tpu_sc as plsc


def _recip(d):
    s = jnp.where(d < 0.0, -1.0, 1.0)
    a = d * s
    sc = jnp.float32(1.0)
    for k in (64, 32, 16, 8, 4, 2, 1):
        hi = jnp.float32(2.0 ** k)
        lo = jnp.float32(2.0 ** -k)
        big = a >= hi
        a = jnp.where(big, a * lo, a)
        sc = jnp.where(big, sc * lo, sc)
    for k in (64, 32, 16, 8, 4, 2, 1):
        hi = jnp.float32(2.0 ** k)
        lo = jnp.float32(2.0 ** -k)
        small = a < jnp.float32(2.0 ** (1 - k))
        a = jnp.where(small, a * hi, a)
        sc = jnp.where(small, sc * hi, sc)
    a = a * 0.5
    sc = sc * 0.5
    r = 2.8235294117647056 - 1.8823529411764706 * a
    for _ in range(4):
        r = r * (2.0 - a * r)
    return r * sc * s


def _sc_body(y_hbm, c_hbm, t_hbm, out_hbm, y_s, c_s, t_s, out_s):
    pltpu.sync_copy(y_hbm, y_s)
    pltpu.sync_copy(c_hbm, c_s)
    pltpu.sync_copy(t_hbm, t_s)
    t = t_s[0]
    out_s[0] = y_s[9] * c_s[2]
    out_s[1] = (y_s[6] + y_s[8]) * c_s[1]
    out_s[2] = (y_s[3] + y_s[5]) * c_s[0]
    out_s[3] = (
        c_s[3]
        + jnp.where(t <= c_s[7], c_s[4], 0.0)
        + c_s[5] * t * _recip(c_s[8])
    )
    pltpu.sync_copy(out_s, out_hbm)


_sc_call = functools.partial(
    pl.kernel,
    mesh=plsc.ScalarSubcoreMesh(axis_name="c", num_cores=2),
    out_type=jax.ShapeDtypeStruct((4,), jnp.float32),
    scratch_types=[
        pltpu.SMEM((10,), jnp.float32),
        pltpu.SMEM((22,), jnp.float32),
        pltpu.SMEM((1,), jnp.float32),
        pltpu.SMEM((4,), jnp.float32),
    ],
)(_sc_body)


@jax.jit
def kernel(y, w, c, t):
    return _sc_call(y, c, t.reshape(1))


# speedup vs baseline: 1.0645x; 1.0645x over previous
import functools

import jax
import jax.numpy as jnp
from jax import lax
from jax.experimental import pallas as pl
from jax.experimental.pallas import tpu as pltpu
from jax.experimental.pallas import tpu_sc as plsc


def _sc_body(y_hbm, c_hbm, t_hbm, out_hbm, y_v, c_v, t_v, out_v, sem):
    @pl.when(lax.axis_index("s") == 0)
    def _():
        d1 = pltpu.async_copy(y_hbm, y_v.at[pl.ds(0, 10)], sem)
        d2 = pltpu.async_copy(c_hbm, c_v.at[pl.ds(0, 22)], sem)
        d3 = pltpu.async_copy(t_hbm, t_v.at[pl.ds(0, 1)], sem)
        d1.wait()
        d2.wait()
        d3.wait()
        yv = y_v[pl.ds(0, 16)]
        cv = c_v[pl.ds(0, 16)]
        t = t_v[pl.ds(0, 16)][0]
        w0 = yv[9] * cv[2]
        w1 = (yv[6] + yv[8]) * cv[1]
        w2 = (yv[3] + yv[5]) * cv[0]
        w3_nodiv = cv[3] + jnp.where(t <= cv[7], cv[4], 0.0)
        num = cv[5] * t
        den = cv[8]
        lane = lax.iota(jnp.int32, 16)
        lane3 = lane == 3
        base = jnp.where(
            lane == 0,
            w0,
            jnp.where(lane == 1, w1, jnp.where(lane == 2, w2, w3_nodiv)),
        )
        res = base + jnp.where(lane3, num, 0.0) / jnp.where(lane3, den, 1.0)
        out_v[...] = res
        pltpu.sync_copy(out_v.at[pl.ds(0, 4)], out_hbm)


_sc_call = functools.partial(
    pl.kernel,
    mesh=plsc.VectorSubcoreMesh(
        core_axis_name="c", subcore_axis_name="s", num_cores=1
    ),
    out_type=jax.ShapeDtypeStruct((4,), jnp.float32),
    scratch_types=[
        pltpu.VMEM((16,), jnp.float32),
        pltpu.VMEM((24,), jnp.float32),
        pltpu.VMEM((16,), jnp.float32),
        pltpu.VMEM((16,), jnp.float32),
        pltpu.SemaphoreType.DMA,
    ],
)(_sc_body)


@jax.jit
def kernel(y, w, c, t):
    return _sc_call(y, c, t.reshape(1))


# speedup vs baseline: 1.0665x; 1.0019x over previous
import functools

import jax
import jax.numpy as jnp
from jax import lax
from jax.experimental import pallas as pl
from jax.experimental.pallas import tpu as pltpu
from jax.experimental.pallas import tpu_sc as plsc


def _sc_body(y_hbm, c_hbm, t_hbm, out_hbm, y_v, c_v, t_v, out_v, sem):
    @pl.when(lax.axis_index("s") == 0)
    def _():
        d1 = pltpu.async_copy(y_hbm, y_v.at[pl.ds(0, 10)], sem)
        d2 = pltpu.async_copy(c_hbm, c_v.at[pl.ds(0, 22)], sem)
        d3 = pltpu.async_copy(t_hbm, t_v.at[pl.ds(0, 1)], sem)
        d1.wait()
        d2.wait()
        d3.wait()
        yv = y_v[pl.ds(0, 16)]
        cv = c_v[pl.ds(0, 16)]
        t = t_v[pl.ds(0, 16)][0]
        w0 = yv[9] * cv[2]
        w1 = (yv[6] + yv[8]) * cv[1]
        w2 = (yv[3] + yv[5]) * cv[0]
        w3_nodiv = cv[3] + jnp.where(t <= cv[7], cv[4], 0.0)
        num = cv[5] * t
        den = cv[8]
        lane = lax.iota(jnp.int32, 16)
        lane3 = lane == 3
        base = jnp.where(
            lane == 0,
            w0,
            jnp.where(lane == 1, w1, jnp.where(lane == 2, w2, w3_nodiv)),
        )
        res = base + jnp.where(lane3, num, 0.0) / jnp.where(lane3, den, 1.0)
        out_v[...] = res
        pltpu.sync_copy(out_v.at[pl.ds(0, 4)], out_hbm)


_sc_call = functools.partial(
    pl.kernel,
    mesh=plsc.VectorSubcoreMesh(
        core_axis_name="c", subcore_axis_name="s", num_cores=1, num_subcores=1
    ),
    out_type=jax.ShapeDtypeStruct((4,), jnp.float32),
    scratch_types=[
        pltpu.VMEM((16,), jnp.float32),
        pltpu.VMEM((24,), jnp.float32),
        pltpu.VMEM((16,), jnp.float32),
        pltpu.VMEM((16,), jnp.float32),
        pltpu.SemaphoreType.DMA,
    ],
)(_sc_body)


@jax.jit
def kernel(y, w, c, t):
    return _sc_call(y, c, t.reshape(1))
